# Initial kernel scaffold; baseline (speedup 1.0000x reference)
import jax
import jax.numpy as jnp
from jax.experimental import pallas as pl


def kernel(features, values, bias, rows, cols):
    raise NotImplementedError("write your pallas kernel here")


# speedup vs baseline: 1.4374x; 1.4374x over previous
import functools

import jax
import jax.numpy as jnp
from jax.experimental import pallas as pl
from jax.experimental.pallas import tpu as pltpu

IN_F = 4096
OUT_F = 4096
NEG_SLOPE = 0.01

BN = 512


def _mm_body(a_ref, b_ref, bias_ref, o_ref):
    acc = jnp.dot(a_ref[...], b_ref[...], preferred_element_type=jnp.float32)
    acc = acc + bias_ref[...]
    o_ref[...] = jnp.where(acc >= 0, acc, NEG_SLOPE * acc)


@functools.partial(jax.jit, donate_argnums=())
def _matmul(features, s, bias2d):
    batch = features.shape[0]
    return pl.pallas_call(
        _mm_body,
        grid=(OUT_F // BN,),
        in_specs=[
            pl.BlockSpec((batch, IN_F), lambda j: (0, 0)),
            pl.BlockSpec((IN_F, BN), lambda j: (0, j)),
            pl.BlockSpec((1, BN), lambda j: (0, j)),
        ],
        out_specs=pl.BlockSpec((batch, BN), lambda j: (0, j)),
        out_shape=jax.ShapeDtypeStruct((batch, OUT_F), jnp.float32),
    )(features, s, bias2d)


def kernel(features, values, bias, rows, cols):
    flat = cols.astype(jnp.int32) * OUT_F + rows.astype(jnp.int32)
    s = (
        jnp.zeros((IN_F * OUT_F,), jnp.float32)
        .at[flat]
        .add(values)
        .reshape(IN_F, OUT_F)
    )
    return _matmul(features, s, bias.reshape(1, OUT_F))
